# Initial kernel scaffold; baseline (speedup 1.0000x reference)
import jax
import jax.numpy as jnp
from jax.experimental import pallas as pl


def kernel(x, original_coord, W_s, W_h, b_h, W_lin, b_lin):
    raise NotImplementedError("write your pallas kernel here")


# speedup vs baseline: 3.9591x; 3.9591x over previous
import jax
import jax.numpy as jnp
from jax.experimental import pallas as pl
from jax.experimental.pallas import tpu as pltpu

_N = 10000
_K = 40
_QB = 256
_NP = 10240
_DIN = 256
_P = 32
_S = 4
_OUT = 32
_RB = 1024


def _prep_body(x_ref, ws_ref, wh_ref, bh_ref, s_ref, h_ref):
    xb = x_ref[...]
    dn = (((1,), (1,)), ((), ()))
    s_ref[...] = jax.lax.dot_general(xb, ws_ref[...], dn,
                                     preferred_element_type=jnp.float32)
    h_ref[...] = jax.lax.dot_general(xb, wh_ref[...], dn,
                                     preferred_element_type=jnp.float32) + bh_ref[...]


def _knn_body(sq_ref, sall_ref, nbr_ref, ew_ref, l1_ref, l2_ref, d2_ref):
    i = pl.program_id(0)
    s_q = sq_ref[...]
    s_all = sall_ref[...]
    dn = (((1,), (1,)), ((), ()))
    sqq = jnp.sum(s_q * s_q, axis=1, keepdims=True)
    ones = jnp.ones((1, _S), jnp.float32)
    sqa = jax.lax.dot_general(ones, s_all * s_all, dn,
                              preferred_element_type=jnp.float32)
    cross = jax.lax.dot_general(s_q, s_all, dn,
                                preferred_element_type=jnp.float32)
    d2 = sqq + sqa - 2.0 * cross
    qid = i * _QB + jax.lax.broadcasted_iota(jnp.int32, (_QB, 1), 0)
    cand = jax.lax.broadcasted_iota(jnp.int32, (_QB, _NP), 1)
    bad = (cand == qid) | (cand >= _N)
    d2_ref[...] = jnp.where(bad, jnp.inf, d2)

    idx_cols = []
    val_cols = []
    for _ in range(_K):
        d = d2_ref[...]
        m = jnp.min(d, axis=1, keepdims=True)
        hit = d == m
        idx = jnp.min(jnp.where(hit, cand, _NP), axis=1, keepdims=True)
        idx_cols.append(idx)
        val_cols.append(m)
        d2_ref[...] = jnp.where(cand == idx, jnp.inf, d)

    nbr = jnp.concatenate(idx_cols, axis=1)
    ew = jnp.maximum(jnp.concatenate(val_cols, axis=1), 0.0)
    nbr_ref[...] = nbr
    ew_ref[...] = ew

    dist = jnp.sqrt(ew + 1e-6)
    sum_dist = jnp.sum(dist, axis=1, keepdims=True)
    valid = qid < _N
    avd = sum_dist / float(_K)
    l1 = jnp.sum(jnp.where(valid, jnp.square(avd - 0.5), 0.0))
    dnorm = dist / (sum_dist + 1e-4)
    snorm = jnp.sum(dnorm, axis=1, keepdims=True)
    gnv = dist / (snorm + 1e-4)
    l2 = jnp.sum(jnp.where(valid, jnp.square(dnorm - gnv), 0.0))
    l1_ref[...] = jnp.full((1, 1, 128), l1, jnp.float32)
    l2_ref[...] = jnp.full((1, 1, 128), l2, jnp.float32)


_SB = 250


def _scatter_body(nbr_ref, ew_ref, h_ref, sum_ref, max_ref, cnt_ref):
    i = pl.program_id(0)

    @pl.when(i == 0)
    def _():
        sum_ref[...] = jnp.zeros((_N, _P), jnp.float32)
        max_ref[...] = jnp.full((_N, _P), -jnp.inf, jnp.float32)
        cnt_ref[...] = jnp.zeros((_N, 8), jnp.float32)

    one_row = jnp.ones((1, 8), jnp.float32)

    def body_q(q, _):
        row = h_ref[pl.ds(i * _SB + q, 1), :]

        def body_k(k, __):
            j = nbr_ref[0, q, k]
            w = ew_ref[0, q, k]
            m = row * w
            sum_ref[pl.ds(j, 1), :] = sum_ref[pl.ds(j, 1), :] + m
            max_ref[pl.ds(j, 1), :] = jnp.maximum(max_ref[pl.ds(j, 1), :], m)
            cnt_ref[pl.ds(j, 1), :] = cnt_ref[pl.ds(j, 1), :] + one_row
            return 0

        jax.lax.fori_loop(0, _K, body_k, 0)
        return 0

    jax.lax.fori_loop(0, _SB, body_q, 0)


def _out_body(sum_ref, max_ref, cnt_ref, x_ref, wl_ref, bl_ref, o_ref):
    c = cnt_ref[...][:, 0:1]
    mean = sum_ref[...] / jnp.maximum(c, 1.0)
    mx = jnp.where(c > 0.0, max_ref[...], 0.0)
    cat = jnp.concatenate([mean, mx, x_ref[...]], axis=1)
    dn = (((1,), (1,)), ((), ()))
    o_ref[...] = jax.lax.dot_general(cat, wl_ref[...], dn,
                                     preferred_element_type=jnp.float32) + bl_ref[...]


def kernel(x, original_coord, W_s, W_h, b_h, W_lin, b_lin):
    del original_coord
    f32 = jnp.float32
    x_pad = jnp.pad(x, ((0, _NP - _N), (0, 0)))
    bh2 = b_h.reshape(1, _P)
    bl2 = b_lin.reshape(1, _OUT)

    s_l_pad, h_pad = pl.pallas_call(
        _prep_body,
        grid=(_NP // _RB,),
        in_specs=[
            pl.BlockSpec((_RB, _DIN), lambda i: (i, 0)),
            pl.BlockSpec((_S, _DIN), lambda i: (0, 0)),
            pl.BlockSpec((_P, _DIN), lambda i: (0, 0)),
            pl.BlockSpec((1, _P), lambda i: (0, 0)),
        ],
        out_specs=[
            pl.BlockSpec((_RB, _S), lambda i: (i, 0)),
            pl.BlockSpec((_RB, _P), lambda i: (i, 0)),
        ],
        out_shape=[
            jax.ShapeDtypeStruct((_NP, _S), f32),
            jax.ShapeDtypeStruct((_NP, _P), f32),
        ],
    )(x_pad, W_s, W_h, bh2)

    nbr, ew, l1p, l2p = pl.pallas_call(
        _knn_body,
        grid=(_NP // _QB,),
        in_specs=[
            pl.BlockSpec((_QB, _S), lambda i: (i, 0)),
            pl.BlockSpec((_NP, _S), lambda i: (0, 0)),
        ],
        out_specs=[
            pl.BlockSpec((_QB, _K), lambda i: (i, 0)),
            pl.BlockSpec((_QB, _K), lambda i: (i, 0)),
            pl.BlockSpec((1, 1, 128), lambda i: (i, 0, 0)),
            pl.BlockSpec((1, 1, 128), lambda i: (i, 0, 0)),
        ],
        out_shape=[
            jax.ShapeDtypeStruct((_NP, _K), jnp.int32),
            jax.ShapeDtypeStruct((_NP, _K), f32),
            jax.ShapeDtypeStruct((_NP // _QB, 1, 128), f32),
            jax.ShapeDtypeStruct((_NP // _QB, 1, 128), f32),
        ],
        scratch_shapes=[pltpu.VMEM((_QB, _NP), f32)],
    )(s_l_pad, s_l_pad)

    sum_r, max_r, cnt_r = pl.pallas_call(
        _scatter_body,
        grid=(_N // _SB,),
        in_specs=[
            pl.BlockSpec((1, _SB, _K), lambda i: (i, 0, 0), memory_space=pltpu.SMEM),
            pl.BlockSpec((1, _SB, _K), lambda i: (i, 0, 0), memory_space=pltpu.SMEM),
            pl.BlockSpec((_N, _P), lambda i: (0, 0)),
        ],
        out_specs=[
            pl.BlockSpec((_N, _P), lambda i: (0, 0)),
            pl.BlockSpec((_N, _P), lambda i: (0, 0)),
            pl.BlockSpec((_N, 8), lambda i: (0, 0)),
        ],
        out_shape=[
            jax.ShapeDtypeStruct((_N, _P), f32),
            jax.ShapeDtypeStruct((_N, _P), f32),
            jax.ShapeDtypeStruct((_N, 8), f32),
        ],
    )(nbr[:_N].reshape(_N // _SB, _SB, _K),
      ew[:_N].reshape(_N // _SB, _SB, _K),
      h_pad[:_N])

    out = pl.pallas_call(
        _out_body,
        grid=(pl.cdiv(_N, _QB),),
        in_specs=[
            pl.BlockSpec((_QB, _P), lambda i: (i, 0)),
            pl.BlockSpec((_QB, _P), lambda i: (i, 0)),
            pl.BlockSpec((_QB, 8), lambda i: (i, 0)),
            pl.BlockSpec((_QB, _DIN), lambda i: (i, 0)),
            pl.BlockSpec((_OUT, _DIN + 2 * _P), lambda i: (0, 0)),
            pl.BlockSpec((1, _OUT), lambda i: (0, 0)),
        ],
        out_specs=pl.BlockSpec((_QB, _OUT), lambda i: (i, 0)),
        out_shape=jax.ShapeDtypeStruct((_N, _OUT), f32),
    )(sum_r, max_r, cnt_r, x, W_lin, bl2)

    s_l = s_l_pad[:_N]
    l1 = 0.01 * (jnp.sum(l1p[:, 0, 0]) / float(_N))
    l2 = 0.1 * (jnp.sum(l2p[:, 0, 0]) / float(_N * _K))
    return (out, s_l, l1, l2)
